# baseline (speedup 1.0000x reference)
import jax
import jax.numpy as jnp
from jax.experimental import pallas as pl
from jax.experimental.pallas import tpu as pltpu

_N, _D, _K, _ITERS = 16384, 256, 64, 11
_C = 1024
_HIGHEST = jax.lax.Precision.HIGHEST


def _xla_lane_reduce(sq):
    def red128(u):
        w = u
        for j in range(1, 16):
            w = w + pltpu.roll(u, 128 - 8 * j, 1)
        for k in (4, 2, 1):
            w = w + pltpu.roll(w, 128 - k, 1)
        return w[:, 0:1]
    return red128(sq[:, :128]) + red128(sq[:, 128:])


def _pick(sa, ia, sb, ib):
    tb = (sb < sa) | ((sb == sa) & (ib < ia))
    return jnp.where(tb, sb, sa), jnp.where(tb, ib, ia)


def _kmeans_body(x_ref, c0_ref, a_ref):
    c = c0_ref[...]
    iota_c = jax.lax.broadcasted_iota(jnp.int32, (_C, _K), 1)
    iota_full = jax.lax.broadcasted_iota(jnp.int32, (_N, _K), 1)
    neg_inf = jnp.float32(-jnp.inf)

    for t in range(_ITERS):
        cn = jnp.transpose(jnp.sum(c * c, axis=1, keepdims=True))

        def chunk_body(j, carry, c=c, cn=cn):
            xj = x_ref[pl.ds(j * _C, _C), :]
            dots = jax.lax.dot_general(xj, c, (((1,), (1,)), ((), ())),
                                       precision=_HIGHEST,
                                       preferred_element_type=jnp.float32)
            proxy = cn - 2.0 * dots
            proxy = jnp.where(jnp.isnan(proxy), neg_inf, proxy)

            def takemin(p):
                m = jnp.min(p, axis=1, keepdims=True)
                i = jnp.min(jnp.where(p == m, iota_c, _K), axis=1,
                            keepdims=True)
                return i, jnp.where(iota_c == i, jnp.float32(jnp.inf), p)

            i1, p2 = takemin(proxy)
            i2, p3 = takemin(p2)
            i3, _ = takemin(p3)

            def exact_dist(idx):
                oh = (iota_c == idx).astype(jnp.float32)
                g = jax.lax.dot_general(oh, c, (((1,), (0,)), ((), ())),
                                        precision=_HIGHEST,
                                        preferred_element_type=jnp.float32)
                dfe = xj - g
                s = _xla_lane_reduce(dfe * dfe)
                return jnp.where(jnp.isnan(s), neg_inf, s)

            s1 = exact_dist(i1)
            s2 = exact_dist(i2)
            s3 = exact_dist(i3)
            s12, i12 = _pick(s1, i1, s2, i2)
            _, aj = _pick(s12, i12, s3, i3)
            a_ref[pl.ds(j * _C, _C), :] = aj
            return carry

        jax.lax.fori_loop(0, _N // _C, chunk_body, jnp.int32(0))

        if t < _ITERS - 1:
            a_full = a_ref[...]
            oh = (iota_full == a_full).astype(jnp.float32)
            xv = x_ref[...]
            sums = jax.lax.dot_general(oh, xv, (((0,), (0,)), ((), ())),
                                       preferred_element_type=jnp.float32)
            counts = jnp.sum(oh, axis=0, keepdims=True)
            c = sums / jnp.transpose(counts)


def kernel(inputs):
    x = inputs
    perm = jax.random.permutation(jax.random.key(1), x.shape[0])
    c0 = jnp.take(x, perm[:_K], axis=0)
    a = pl.pallas_call(
        _kmeans_body,
        out_shape=jax.ShapeDtypeStruct((_N, 1), jnp.int32),
    )(x, c0)
    return a[:, 0].astype(jnp.int64)
